# baseline (speedup 1.0000x reference)
import jax
import jax.numpy as jnp
from jax import lax
from jax.experimental import pallas as pl
from jax.experimental.pallas import tpu as pltpu

_B, _S, _K = 32, 256, 8192
_RB = 16


def _rotl(x, d):
    return lax.shift_left(x, jnp.int32(d)) | lax.shift_right_logical(
        x, jnp.int32(32 - d)
    )


def _threefry2x32(x0, x1):
    ks0 = jnp.int32(0)
    ks1 = jnp.int32(42)
    ks2 = ks0 ^ ks1 ^ jnp.int32(0x1BD11BDA)
    ks = (ks0, ks1, ks2)
    rots = ((13, 15, 26, 6), (17, 29, 16, 24))
    x0 = x0 + ks0
    x1 = x1 + ks1
    for j in range(1, 6):
        for r in rots[(j - 1) % 2]:
            x0 = x0 + x1
            x1 = _rotl(x1, r)
            x1 = x0 ^ x1
        x0 = x0 + ks[j % 3]
        x1 = x1 + ks[(j + 1) % 3] + jnp.int32(j)
    return x0, x1


def _gumbel_from_bits(bits):
    fb = lax.shift_right_logical(bits, jnp.int32(9)) | jnp.int32(0x3F800000)
    u = lax.bitcast_convert_type(fb, jnp.float32) - jnp.float32(1.0)
    return -jnp.log(-jnp.log(u + 1e-20) + 1e-20)


def _body(x_ref, o_ref):
    b = pl.program_id(0)
    j = pl.program_id(1)
    base = (b * _S + j * _RB) * _K
    rr = lax.broadcasted_iota(jnp.int32, (_RB, _K), 0)
    kk = lax.broadcasted_iota(jnp.int32, (_RB, _K), 1)
    p = base + rr * _K + kk
    o0, o1 = _threefry2x32(jnp.zeros_like(p), p)
    g = _gumbel_from_bits(o0 ^ o1)
    z = x_ref[0] + g
    m = jnp.max(z, axis=-1, keepdims=True)
    cand = jnp.where(z == m, kk, jnp.int32(_K))
    idx = jnp.min(cand, axis=-1, keepdims=True)
    o_ref[0] = jnp.where(kk == idx, jnp.float32(1.0), jnp.float32(0.0))


@jax.jit
def kernel(logits):
    return pl.pallas_call(
        _body,
        grid=(_B, _S // _RB),
        in_specs=[pl.BlockSpec((1, _RB, _K), lambda b, j: (b, j, 0))],
        out_specs=pl.BlockSpec((1, _RB, _K), lambda b, j: (b, j, 0)),
        out_shape=jax.ShapeDtypeStruct((_B, _S, _K), jnp.float32),
        compiler_params=pltpu.CompilerParams(
            dimension_semantics=("parallel", "parallel")
        ),
    )(logits)


# speedup vs baseline: 1.4095x; 1.4095x over previous
import jax
import jax.numpy as jnp
from jax import lax
from jax.experimental import pallas as pl
from jax.experimental.pallas import tpu as pltpu

_B, _S, _K = 32, 256, 8192
_RB = 16


def _rotl(x, d):
    return lax.shift_left(x, jnp.int32(d)) | lax.shift_right_logical(
        x, jnp.int32(32 - d)
    )


def _threefry2x32(x0, x1):
    ks0 = jnp.int32(0)
    ks1 = jnp.int32(42)
    ks2 = ks0 ^ ks1 ^ jnp.int32(0x1BD11BDA)
    ks = (ks0, ks1, ks2)
    rots = ((13, 15, 26, 6), (17, 29, 16, 24))
    x0 = x0 + ks0
    x1 = x1 + ks1
    for j in range(1, 6):
        for r in rots[(j - 1) % 2]:
            x0 = x0 + x1
            x1 = _rotl(x1, r)
            x1 = x0 ^ x1
        x0 = x0 + ks[j % 3]
        x1 = x1 + ks[(j + 1) % 3] + jnp.int32(j)
    return x0, x1


def _gumbel_from_bits(bits):
    fb = lax.shift_right_logical(bits, jnp.int32(9)) | jnp.int32(0x3F800000)
    u = lax.bitcast_convert_type(fb, jnp.float32) - jnp.float32(1.0)
    return -jnp.log(-jnp.log(u + 1e-20) + 1e-20)


_CW = 1024


def _body(x_ref, o_ref):
    b = pl.program_id(0)
    j = pl.program_id(1)
    base = (b * _S + j * _RB) * _K
    rr = lax.broadcasted_iota(jnp.int32, (_RB, _CW), 0)
    kk = lax.broadcasted_iota(jnp.int32, (_RB, _CW), 1)
    p0 = base + rr * _K + kk
    m = jnp.full((_RB, 1), -jnp.inf, jnp.float32)
    idx = jnp.zeros((_RB, 1), jnp.int32)
    for c in range(_K // _CW):
        p = p0 + jnp.int32(c * _CW)
        o0, o1 = _threefry2x32(jnp.zeros_like(p), p)
        g = _gumbel_from_bits(o0 ^ o1)
        z = x_ref[0, :, c * _CW:(c + 1) * _CW] + g
        mc = jnp.max(z, axis=-1, keepdims=True)
        cand = jnp.where(z == mc, kk, jnp.int32(_CW))
        ic = jnp.min(cand, axis=-1, keepdims=True) + jnp.int32(c * _CW)
        better = mc > m
        m = jnp.where(better, mc, m)
        idx = jnp.where(better, ic, idx)
    for c in range(_K // _CW):
        o_ref[0, :, c * _CW:(c + 1) * _CW] = jnp.where(
            kk + jnp.int32(c * _CW) == idx, jnp.float32(1.0), jnp.float32(0.0)
        )


@jax.jit
def kernel(logits):
    return pl.pallas_call(
        _body,
        grid=(_B, _S // _RB),
        in_specs=[pl.BlockSpec((1, _RB, _K), lambda b, j: (b, j, 0))],
        out_specs=pl.BlockSpec((1, _RB, _K), lambda b, j: (b, j, 0)),
        out_shape=jax.ShapeDtypeStruct((_B, _S, _K), jnp.float32),
        compiler_params=pltpu.CompilerParams(
            dimension_semantics=("parallel", "parallel")
        ),
    )(logits)


# speedup vs baseline: 1.4989x; 1.0634x over previous
import jax
import jax.numpy as jnp
from jax import lax
from jax.experimental import pallas as pl
from jax.experimental.pallas import tpu as pltpu

_B, _S, _K = 32, 256, 8192
_RB = 64


def _rotl(x, d):
    return lax.shift_left(x, jnp.int32(d)) | lax.shift_right_logical(
        x, jnp.int32(32 - d)
    )


def _threefry2x32(x0, x1):
    ks0 = jnp.int32(0)
    ks1 = jnp.int32(42)
    ks2 = ks0 ^ ks1 ^ jnp.int32(0x1BD11BDA)
    ks = (ks0, ks1, ks2)
    rots = ((13, 15, 26, 6), (17, 29, 16, 24))
    x0 = x0 + ks0
    x1 = x1 + ks1
    for j in range(1, 6):
        for r in rots[(j - 1) % 2]:
            x0 = x0 + x1
            x1 = _rotl(x1, r)
            x1 = x0 ^ x1
        x0 = x0 + ks[j % 3]
        x1 = x1 + ks[(j + 1) % 3] + jnp.int32(j)
    return x0, x1


def _gumbel_from_bits(bits):
    fb = lax.shift_right_logical(bits, jnp.int32(9)) | jnp.int32(0x3F800000)
    u = lax.bitcast_convert_type(fb, jnp.float32) - jnp.float32(1.0)
    return -jnp.log(-jnp.log(u + 1e-20) + 1e-20)


_CW = 1024


def _body(x_ref, o_ref):
    b = pl.program_id(0)
    j = pl.program_id(1)
    base = (b * _S + j * _RB) * _K
    rr = lax.broadcasted_iota(jnp.int32, (_RB, _CW), 0)
    kk = lax.broadcasted_iota(jnp.int32, (_RB, _CW), 1)
    p0 = base + rr * _K + kk
    m = jnp.full((_RB, 1), -jnp.inf, jnp.float32)
    idx = jnp.zeros((_RB, 1), jnp.int32)
    for c in range(_K // _CW):
        p = p0 + jnp.int32(c * _CW)
        o0, o1 = _threefry2x32(jnp.zeros_like(p), p)
        g = _gumbel_from_bits(o0 ^ o1)
        z = x_ref[0, :, c * _CW:(c + 1) * _CW] + g
        mc = jnp.max(z, axis=-1, keepdims=True)
        cand = jnp.where(z == mc, kk, jnp.int32(_CW))
        ic = jnp.min(cand, axis=-1, keepdims=True) + jnp.int32(c * _CW)
        better = mc > m
        m = jnp.where(better, mc, m)
        idx = jnp.where(better, ic, idx)
    for c in range(_K // _CW):
        o_ref[0, :, c * _CW:(c + 1) * _CW] = jnp.where(
            kk + jnp.int32(c * _CW) == idx, jnp.float32(1.0), jnp.float32(0.0)
        )


@jax.jit
def kernel(logits):
    return pl.pallas_call(
        _body,
        grid=(_B, _S // _RB),
        in_specs=[pl.BlockSpec((1, _RB, _K), lambda b, j: (b, j, 0))],
        out_specs=pl.BlockSpec((1, _RB, _K), lambda b, j: (b, j, 0)),
        out_shape=jax.ShapeDtypeStruct((_B, _S, _K), jnp.float32),
        compiler_params=pltpu.CompilerParams(
            dimension_semantics=("parallel", "parallel")
        ),
    )(logits)
